# Initial kernel scaffold; baseline (speedup 1.0000x reference)
import jax
import jax.numpy as jnp
from jax.experimental import pallas as pl


def kernel(x, edge_index, W1, W2):
    raise NotImplementedError("write your pallas kernel here")


# speedup vs baseline: 14.0512x; 14.0512x over previous
import functools
import math

import jax
import jax.numpy as jnp
from jax import lax
from jax.experimental import pallas as pl
from jax.experimental.pallas import tpu as pltpu
from jax.experimental.pallas import tpu_sc as plsc

N = 10000
D = 128
E = 320000
L = 4
ALPHA = 0.5
THETA = 1.0

NC = 2
NS = 16
LANES = 16
NW = NC * NS
EPW = E // NW
K = 80
NCH = EPW // K
NPAD = 10240
RPT = NPAD // NS

_mesh = plsc.VectorSubcoreMesh(
    core_axis_name="c", subcore_axis_name="s", num_cores=NC, num_subcores=NS
)


def _deg_body(col_hbm, z16_hbm, out_hbm, cidx_v, ones_v, deg_sh):
    core = lax.axis_index("c")
    sid = lax.axis_index("s")
    wid = sid * NC + core

    pltpu.sync_copy(col_hbm.at[wid], cidx_v)

    @pl.loop(0, K)
    def _(i):
        ones_v.at[i][...] = jnp.full((LANES,), 1.0, jnp.float32)

    pltpu.sync_copy(
        z16_hbm.at[pl.ds(sid * RPT, RPT)], deg_sh.at[pl.ds(sid * RPT, RPT)]
    )
    plsc.subcore_barrier()

    @pl.loop(0, NCH)
    def _(j):
        pltpu.sync_copy(ones_v, deg_sh.at[cidx_v.at[j]], add=True)

    plsc.subcore_barrier()
    pltpu.sync_copy(
        deg_sh.at[pl.ds(sid * RPT, RPT)],
        out_hbm.at[core, pl.ds(sid * RPT, RPT)],
    )


_deg_kernel = pl.kernel(
    _deg_body,
    out_type=jax.ShapeDtypeStruct((NC, NPAD, LANES), jnp.float32),
    mesh=_mesh,
    scratch_types=[
        pltpu.VMEM((NCH, K), jnp.int32),
        pltpu.VMEM((K, LANES), jnp.float32),
        pltpu.VMEM_SHARED((NPAD, LANES), jnp.float32),
    ],
)


def _scatter_body(g_hbm, row_hbm, col_hbm, z_hbm, out_hbm, ridx_v, cidx_v, rows_v, agg_sh):
    core = lax.axis_index("c")
    sid = lax.axis_index("s")
    wid = sid * NC + core

    pltpu.sync_copy(row_hbm.at[wid], ridx_v)
    pltpu.sync_copy(col_hbm.at[wid], cidx_v)
    pltpu.sync_copy(
        z_hbm.at[pl.ds(sid * RPT, RPT)], agg_sh.at[pl.ds(sid * RPT, RPT)]
    )
    plsc.subcore_barrier()

    @pl.loop(0, NCH)
    def _(j):
        pltpu.sync_copy(g_hbm.at[ridx_v.at[j]], rows_v)
        pltpu.sync_copy(rows_v, agg_sh.at[cidx_v.at[j]], add=True)

    plsc.subcore_barrier()
    pltpu.sync_copy(
        agg_sh.at[pl.ds(sid * RPT, RPT)],
        out_hbm.at[core, pl.ds(sid * RPT, RPT)],
    )


_scatter_kernel = pl.kernel(
    _scatter_body,
    out_type=jax.ShapeDtypeStruct((NC, NPAD, D), jnp.float32),
    mesh=_mesh,
    scratch_types=[
        pltpu.VMEM((NCH, K), jnp.int32),
        pltpu.VMEM((NCH, K), jnp.int32),
        pltpu.VMEM((K, D), jnp.float32),
        pltpu.VMEM_SHARED((NPAD, D), jnp.float32),
    ],
)


BN = 1000


def _init_body(x_ref, degp_ref, f0_ref, g0_ref, dinvb_ref):
    degp = degp_ref[...]
    deg = degp[0, :, 0:1] + degp[1, :, 0:1] + 1.0
    dinvb = jnp.broadcast_to(lax.rsqrt(deg), (BN, D))
    f0 = jnp.maximum(x_ref[...], 0.0)
    f0_ref[...] = f0
    g0_ref[...] = f0 * dinvb
    dinvb_ref[...] = dinvb


def _init_call(x, degp):
    return pl.pallas_call(
        _init_body,
        grid=(N // BN,),
        in_specs=[
            pl.BlockSpec((BN, D), lambda i: (i, 0)),
            pl.BlockSpec((NC, BN, LANES), lambda i: (0, i, 0)),
        ],
        out_specs=[
            pl.BlockSpec((BN, D), lambda i: (i, 0)),
            pl.BlockSpec((BN, D), lambda i: (i, 0)),
            pl.BlockSpec((BN, D), lambda i: (i, 0)),
        ],
        out_shape=[
            jax.ShapeDtypeStruct((N, D), jnp.float32),
            jax.ShapeDtypeStruct((N, D), jnp.float32),
            jax.ShapeDtypeStruct((N, D), jnp.float32),
        ],
    )(x, degp)


def _dense_body(beta, s_ref, g_ref, f0_ref, dinvb_ref, w1_ref, w2_ref, f_ref, gn_ref):
    dinvb = dinvb_ref[...]
    agg = dinvb * (s_ref[0] + s_ref[1] + g_ref[...])
    h = (1.0 - ALPHA) * agg
    x0 = ALPHA * f0_ref[...]
    out = (1.0 - beta) * h + beta * jnp.dot(
        h, w1_ref[...], preferred_element_type=jnp.float32
    )
    out = out + (1.0 - beta) * x0 + beta * jnp.dot(
        x0, w2_ref[...], preferred_element_type=jnp.float32
    )
    f = jnp.maximum(out, 0.0)
    f_ref[...] = f
    gn_ref[...] = f * dinvb


def _dense_call(beta, S, g, f0, dinvb, w1, w2):
    return pl.pallas_call(
        functools.partial(_dense_body, beta),
        grid=(N // BN,),
        in_specs=[
            pl.BlockSpec((NC, BN, D), lambda i: (0, i, 0)),
            pl.BlockSpec((BN, D), lambda i: (i, 0)),
            pl.BlockSpec((BN, D), lambda i: (i, 0)),
            pl.BlockSpec((BN, D), lambda i: (i, 0)),
            pl.BlockSpec((D, D), lambda i: (0, 0)),
            pl.BlockSpec((D, D), lambda i: (0, 0)),
        ],
        out_specs=[
            pl.BlockSpec((BN, D), lambda i: (i, 0)),
            pl.BlockSpec((BN, D), lambda i: (i, 0)),
        ],
        out_shape=[
            jax.ShapeDtypeStruct((N, D), jnp.float32),
            jax.ShapeDtypeStruct((N, D), jnp.float32),
        ],
    )(S, g, f0, dinvb, w1, w2)


def kernel(x, edge_index, W1, W2):
    row2 = edge_index[0].reshape(NW, NCH, K)
    col2 = edge_index[1].reshape(NW, NCH, K)
    z16 = jnp.zeros((NPAD, LANES), jnp.float32)
    z128 = jnp.zeros((NPAD, D), jnp.float32)

    degp = _deg_kernel(col2, z16)
    f0, g, dinvb = _init_call(x, degp)
    f = f0
    for i in range(L):
        S = _scatter_kernel(g, row2, col2, z128)
        beta = float(math.log(THETA / (i + 1) + 1.0))
        f, g = _dense_call(beta, S, g, f0, dinvb, W1[i], W2[i])
    return f


# speedup vs baseline: 18.1400x; 1.2910x over previous
import functools
import math

import jax
import jax.numpy as jnp
from jax import lax
from jax.experimental import pallas as pl
from jax.experimental.pallas import tpu as pltpu
from jax.experimental.pallas import tpu_sc as plsc

N = 10000
D = 128
E = 320000
L = 4
ALPHA = 0.5
THETA = 1.0

NC = 2
NS = 16
LANES = 16
NW = NC * NS
EPW = E // NW
K = 128
CH = 80
EPT = CH * K
PH = 2
CHP = CH // PH
R = 2
NBR = CHP // R
NPAD = 10240
RPT = NPAD // NS

_mesh = plsc.VectorSubcoreMesh(
    core_axis_name="c", subcore_axis_name="s", num_cores=NC, num_subcores=NS
)


DK = 80
DNCH = EPW // DK


def _deg_body(col_hbm, ones_hbm, z16_hbm, out_hbm, cidx_v, ones_v, deg_sh):
    core = lax.axis_index("c")
    sid = lax.axis_index("s")
    wid = sid * NC + core

    pltpu.sync_copy(col_hbm.at[wid], cidx_v)
    pltpu.sync_copy(ones_hbm, ones_v)
    pltpu.sync_copy(
        z16_hbm.at[pl.ds(sid * RPT, RPT)], deg_sh.at[pl.ds(sid * RPT, RPT)]
    )
    plsc.subcore_barrier()

    @pl.loop(0, DNCH)
    def _(j):
        pltpu.sync_copy(ones_v, deg_sh.at[cidx_v.at[j]], add=True)

    plsc.subcore_barrier()
    pltpu.sync_copy(
        deg_sh.at[pl.ds(sid * RPT, RPT)],
        out_hbm.at[core, pl.ds(sid * RPT, RPT)],
    )


_deg_kernel = pl.kernel(
    _deg_body,
    out_type=jax.ShapeDtypeStruct((NC, NPAD, LANES), jnp.float32),
    mesh=_mesh,
    compiler_params=pltpu.CompilerParams(use_tc_tiling_on_sc=False),
    scratch_types=[
        pltpu.VMEM((DNCH, DK), jnp.int32),
        pltpu.VMEM((DK, LANES), jnp.float32),
        pltpu.VMEM_SHARED((NPAD, LANES), jnp.float32),
    ],
)


def _scatter_body(g_hbm, row_hbm, col_hbm, z_hbm, out_hbm,
                  ridx_v, cidx_v, rows_v, agg_sh, *sems):
    gsems, ssems = sems[:R], sems[R:]
    core = lax.axis_index("c")
    sid = lax.axis_index("s")
    wid = sid * NC + core

    pltpu.sync_copy(
        z_hbm.at[pl.ds(sid * RPT, RPT)], agg_sh.at[pl.ds(sid * RPT, RPT)]
    )
    plsc.subcore_barrier()

    for p in range(PH):
        pltpu.sync_copy(row_hbm.at[wid, pl.ds(p * CHP, CHP)], ridx_v)
        pltpu.sync_copy(col_hbm.at[wid, pl.ds(p * CHP, CHP)], cidx_v)

        @pl.loop(0, NBR)
        def _(t):
            base = t * R
            for r in range(R):
                pltpu.async_copy(
                    g_hbm.at[ridx_v.at[base + r]], rows_v.at[r], gsems[r]
                )
            for r in range(R):
                pltpu.make_async_copy(
                    g_hbm.at[ridx_v.at[base + r]], rows_v.at[r], gsems[r]
                ).wait()
                pltpu.sync_copy(
                    rows_v.at[r], agg_sh.at[cidx_v.at[base + r]], add=True
                )

    plsc.subcore_barrier()
    pltpu.sync_copy(
        agg_sh.at[pl.ds(sid * RPT, RPT)],
        out_hbm.at[core, pl.ds(sid * RPT, RPT)],
    )


_scatter_kernel = pl.kernel(
    _scatter_body,
    out_type=jax.ShapeDtypeStruct((NC, NPAD, D), jnp.float32),
    mesh=_mesh,
    scratch_types=[
        pltpu.VMEM((CHP, K), jnp.int32),
        pltpu.VMEM((CHP, K), jnp.int32),
        pltpu.VMEM((R, K, D), jnp.float32),
        pltpu.VMEM_SHARED((NPAD, D), jnp.float32),
    ] + [pltpu.SemaphoreType.DMA] * (2 * R),
)


BN = 1000


def _init_body(x_ref, degp_ref, f0_ref, g0_ref, dinvb_ref):
    degp = degp_ref[...]
    deg = degp[0, :, 0:1] + degp[1, :, 0:1] + 1.0
    dinvb = jnp.broadcast_to(lax.rsqrt(deg), (BN, D))
    f0 = jnp.maximum(x_ref[...], 0.0)
    f0_ref[...] = f0
    g0_ref[...] = f0 * dinvb
    dinvb_ref[...] = dinvb


def _init_call(x, degp):
    return pl.pallas_call(
        _init_body,
        grid=(N // BN,),
        in_specs=[
            pl.BlockSpec((BN, D), lambda i: (i, 0)),
            pl.BlockSpec((NC, BN, LANES), lambda i: (0, i, 0)),
        ],
        out_specs=[
            pl.BlockSpec((BN, D), lambda i: (i, 0)),
            pl.BlockSpec((BN, D), lambda i: (i, 0)),
            pl.BlockSpec((BN, D), lambda i: (i, 0)),
        ],
        out_shape=[
            jax.ShapeDtypeStruct((N, D), jnp.float32),
            jax.ShapeDtypeStruct((N, D), jnp.float32),
            jax.ShapeDtypeStruct((N, D), jnp.float32),
        ],
    )(x, degp)


def _dense_body(beta, s_ref, g_ref, f0_ref, dinvb_ref, w1_ref, w2_ref, f_ref, gn_ref):
    dinvb = dinvb_ref[...]
    agg = dinvb * (s_ref[0] + s_ref[1] + g_ref[...])
    h = (1.0 - ALPHA) * agg
    x0 = ALPHA * f0_ref[...]
    out = (1.0 - beta) * h + beta * jnp.dot(
        h, w1_ref[...], preferred_element_type=jnp.float32
    )
    out = out + (1.0 - beta) * x0 + beta * jnp.dot(
        x0, w2_ref[...], preferred_element_type=jnp.float32
    )
    f = jnp.maximum(out, 0.0)
    f_ref[...] = f
    gn_ref[...] = f * dinvb


def _dense_call(beta, S, g, f0, dinvb, w1, w2):
    return pl.pallas_call(
        functools.partial(_dense_body, beta),
        grid=(N // BN,),
        in_specs=[
            pl.BlockSpec((NC, BN, D), lambda i: (0, i, 0)),
            pl.BlockSpec((BN, D), lambda i: (i, 0)),
            pl.BlockSpec((BN, D), lambda i: (i, 0)),
            pl.BlockSpec((BN, D), lambda i: (i, 0)),
            pl.BlockSpec((D, D), lambda i: (0, 0)),
            pl.BlockSpec((D, D), lambda i: (0, 0)),
        ],
        out_specs=[
            pl.BlockSpec((BN, D), lambda i: (i, 0)),
            pl.BlockSpec((BN, D), lambda i: (i, 0)),
        ],
        out_shape=[
            jax.ShapeDtypeStruct((N, D), jnp.float32),
            jax.ShapeDtypeStruct((N, D), jnp.float32),
        ],
    )(S, g, f0, dinvb, w1, w2)


def kernel(x, edge_index, W1, W2):
    row_r = edge_index[0].reshape(NW, EPW)
    col_r = edge_index[1].reshape(NW, EPW)
    fi = lax.broadcasted_iota(jnp.int32, (NW, EPT - EPW), 1)
    fake_row = (fi * 41) % N
    fake_col = N + fi % (NPAD - N)
    row3 = jnp.concatenate([row_r, fake_row], axis=1).reshape(NW, CH, K)
    col3 = jnp.concatenate([col_r, fake_col], axis=1).reshape(NW, CH, K)
    z16 = jnp.zeros((NPAD, LANES), jnp.float32)
    z128 = jnp.zeros((NPAD, D), jnp.float32)

    col2 = edge_index[1].reshape(NW, DNCH, DK)
    ones16 = jnp.ones((DK, LANES), jnp.float32)
    degp = _deg_kernel(col2, ones16, z16)
    f0, g, dinvb = _init_call(x, degp)
    f = f0
    for i in range(L):
        S = _scatter_kernel(g, row3, col3, z128)
        beta = float(math.log(THETA / (i + 1) + 1.0))
        f, g = _dense_call(beta, S, g, f0, dinvb, W1[i], W2[i])
    return f


# speedup vs baseline: 18.6425x; 1.0277x over previous
import functools
import math

import jax
import jax.numpy as jnp
from jax import lax
from jax.experimental import pallas as pl
from jax.experimental.pallas import tpu as pltpu
from jax.experimental.pallas import tpu_sc as plsc

N = 10000
D = 128
E = 320000
L = 4
ALPHA = 0.5
THETA = 1.0

NC = 2
NS = 16
LANES = 16
NW = NC * NS
EPW = E // NW
K = 128
CH = 80
EPT = CH * K
PH = 2
CHP = CH // PH
R = 2
NBR = CHP // R
NPAD = 10240
RPT = NPAD // NS

_mesh = plsc.VectorSubcoreMesh(
    core_axis_name="c", subcore_axis_name="s", num_cores=NC, num_subcores=NS
)


DK = 80
DNCH = EPW // DK


def _deg_body(col_hbm, ones_hbm, z16_hbm, out_hbm, cidx_v, ones_v, deg_sh):
    core = lax.axis_index("c")
    sid = lax.axis_index("s")
    wid = sid * NC + core

    pltpu.sync_copy(col_hbm.at[wid], cidx_v)
    pltpu.sync_copy(ones_hbm, ones_v)
    pltpu.sync_copy(
        z16_hbm.at[pl.ds(sid * RPT, RPT)], deg_sh.at[pl.ds(sid * RPT, RPT)]
    )
    plsc.subcore_barrier()

    @pl.loop(0, DNCH)
    def _(j):
        pltpu.sync_copy(ones_v, deg_sh.at[cidx_v.at[j]], add=True)

    plsc.subcore_barrier()
    pltpu.sync_copy(
        deg_sh.at[pl.ds(sid * RPT, RPT)],
        out_hbm.at[core, pl.ds(sid * RPT, RPT)],
    )


_deg_kernel = pl.kernel(
    _deg_body,
    out_type=jax.ShapeDtypeStruct((NC, NPAD, LANES), jnp.float32),
    mesh=_mesh,
    compiler_params=pltpu.CompilerParams(use_tc_tiling_on_sc=False),
    scratch_types=[
        pltpu.VMEM((DNCH, DK), jnp.int32),
        pltpu.VMEM((DK, LANES), jnp.float32),
        pltpu.VMEM_SHARED((NPAD, LANES), jnp.float32),
    ],
)


def _scatter_body(g_hbm, row_hbm, col_hbm, z_hbm, out_hbm,
                  ridx_v, cidx_v, rows_v, agg_sh, *sems):
    gsems, ssems = sems[:R], sems[R:]
    core = lax.axis_index("c")
    sid = lax.axis_index("s")
    wid = sid * NC + core

    pltpu.sync_copy(
        z_hbm.at[pl.ds(sid * RPT, RPT)], agg_sh.at[pl.ds(sid * RPT, RPT)]
    )
    plsc.subcore_barrier()

    for p in range(PH):
        pltpu.sync_copy(row_hbm.at[wid, pl.ds(p * CHP, CHP)], ridx_v)
        pltpu.sync_copy(col_hbm.at[wid, pl.ds(p * CHP, CHP)], cidx_v)

        @pl.loop(0, NBR)
        def _(t):
            base = t * R
            for r in range(R):
                @pl.when(t > 0)
                def _():
                    pltpu.make_async_copy(
                        rows_v.at[r], agg_sh.at[cidx_v.at[base - R + r]],
                        ssems[r],
                    ).wait()

                pltpu.async_copy(
                    g_hbm.at[ridx_v.at[base + r]], rows_v.at[r], gsems[r]
                )
            for r in range(R):
                pltpu.make_async_copy(
                    g_hbm.at[ridx_v.at[base + r]], rows_v.at[r], gsems[r]
                ).wait()
                pltpu.async_copy(
                    rows_v.at[r], agg_sh.at[cidx_v.at[base + r]], ssems[r],
                    add=True,
                )

        for r in range(R):
            pltpu.make_async_copy(
                rows_v.at[r], agg_sh.at[cidx_v.at[CHP - R + r]], ssems[r]
            ).wait()

    plsc.subcore_barrier()
    pltpu.sync_copy(
        agg_sh.at[pl.ds(sid * RPT, RPT)],
        out_hbm.at[core, pl.ds(sid * RPT, RPT)],
    )


_scatter_kernel = pl.kernel(
    _scatter_body,
    out_type=jax.ShapeDtypeStruct((NC, NPAD, D), jnp.float32),
    mesh=_mesh,
    scratch_types=[
        pltpu.VMEM((CHP, K), jnp.int32),
        pltpu.VMEM((CHP, K), jnp.int32),
        pltpu.VMEM((R, K, D), jnp.float32),
        pltpu.VMEM_SHARED((NPAD, D), jnp.float32),
    ] + [pltpu.SemaphoreType.DMA] * (2 * R),
)


BN = 1000


def _init_body(x_ref, degp_ref, f0_ref, g0_ref, dinvb_ref):
    degp = degp_ref[...]
    deg = degp[0, :, 0:1] + degp[1, :, 0:1] + 1.0
    dinvb = jnp.broadcast_to(lax.rsqrt(deg), (BN, D))
    f0 = jnp.maximum(x_ref[...], 0.0)
    f0_ref[...] = f0
    g0_ref[...] = f0 * dinvb
    dinvb_ref[...] = dinvb


def _init_call(x, degp):
    return pl.pallas_call(
        _init_body,
        grid=(N // BN,),
        in_specs=[
            pl.BlockSpec((BN, D), lambda i: (i, 0)),
            pl.BlockSpec((NC, BN, LANES), lambda i: (0, i, 0)),
        ],
        out_specs=[
            pl.BlockSpec((BN, D), lambda i: (i, 0)),
            pl.BlockSpec((BN, D), lambda i: (i, 0)),
            pl.BlockSpec((BN, D), lambda i: (i, 0)),
        ],
        out_shape=[
            jax.ShapeDtypeStruct((N, D), jnp.float32),
            jax.ShapeDtypeStruct((N, D), jnp.float32),
            jax.ShapeDtypeStruct((N, D), jnp.float32),
        ],
    )(x, degp)


def _dense_body(beta, s_ref, g_ref, f0_ref, dinvb_ref, w1_ref, w2_ref, f_ref, gn_ref):
    dinvb = dinvb_ref[...]
    agg = dinvb * (s_ref[0] + s_ref[1] + g_ref[...])
    h = (1.0 - ALPHA) * agg
    x0 = ALPHA * f0_ref[...]
    out = (1.0 - beta) * h + beta * jnp.dot(
        h, w1_ref[...], preferred_element_type=jnp.float32
    )
    out = out + (1.0 - beta) * x0 + beta * jnp.dot(
        x0, w2_ref[...], preferred_element_type=jnp.float32
    )
    f = jnp.maximum(out, 0.0)
    f_ref[...] = f
    gn_ref[...] = f * dinvb


def _dense_call(beta, S, g, f0, dinvb, w1, w2):
    return pl.pallas_call(
        functools.partial(_dense_body, beta),
        grid=(N // BN,),
        in_specs=[
            pl.BlockSpec((NC, BN, D), lambda i: (0, i, 0)),
            pl.BlockSpec((BN, D), lambda i: (i, 0)),
            pl.BlockSpec((BN, D), lambda i: (i, 0)),
            pl.BlockSpec((BN, D), lambda i: (i, 0)),
            pl.BlockSpec((D, D), lambda i: (0, 0)),
            pl.BlockSpec((D, D), lambda i: (0, 0)),
        ],
        out_specs=[
            pl.BlockSpec((BN, D), lambda i: (i, 0)),
            pl.BlockSpec((BN, D), lambda i: (i, 0)),
        ],
        out_shape=[
            jax.ShapeDtypeStruct((N, D), jnp.float32),
            jax.ShapeDtypeStruct((N, D), jnp.float32),
        ],
    )(S, g, f0, dinvb, w1, w2)


def kernel(x, edge_index, W1, W2):
    row_r = edge_index[0].reshape(NW, EPW)
    col_r = edge_index[1].reshape(NW, EPW)
    fi = lax.broadcasted_iota(jnp.int32, (NW, EPT - EPW), 1)
    fake_row = (fi * 41) % N
    fake_col = N + fi % (NPAD - N)
    row3 = jnp.concatenate([row_r, fake_row], axis=1).reshape(NW, CH, K)
    col3 = jnp.concatenate([col_r, fake_col], axis=1).reshape(NW, CH, K)
    z16 = jnp.zeros((NPAD, LANES), jnp.float32)
    z128 = jnp.zeros((NPAD, D), jnp.float32)

    col2 = edge_index[1].reshape(NW, DNCH, DK)
    ones16 = jnp.ones((DK, LANES), jnp.float32)
    degp = _deg_kernel(col2, ones16, z16)
    f0, g, dinvb = _init_call(x, degp)
    f = f0
    for i in range(L):
        S = _scatter_kernel(g, row3, col3, z128)
        beta = float(math.log(THETA / (i + 1) + 1.0))
        f, g = _dense_call(beta, S, g, f0, dinvb, W1[i], W2[i])
    return f


# speedup vs baseline: 18.8368x; 1.0104x over previous
import functools
import math

import jax
import jax.numpy as jnp
from jax import lax
from jax.experimental import pallas as pl
from jax.experimental.pallas import tpu as pltpu
from jax.experimental.pallas import tpu_sc as plsc

N = 10000
D = 128
E = 320000
L = 4
ALPHA = 0.5
THETA = 1.0

NC = 2
NS = 16
LANES = 16
NW = NC * NS
EPW = E // NW
K = 128
CH = 80
EPT = CH * K
PH = 2
CHP = CH // PH
R = 2
NBR = CHP // R
NPAD = 10240
RPT = NPAD // NS

_mesh = plsc.VectorSubcoreMesh(
    core_axis_name="c", subcore_axis_name="s", num_cores=NC, num_subcores=NS
)


DK = 80
DNCH = EPW // DK


def _deg_body(col_hbm, ones_hbm, z16_hbm, out_hbm, cidx_v, ones_v, deg_sh):
    core = lax.axis_index("c")
    sid = lax.axis_index("s")
    wid = sid * NC + core

    pltpu.sync_copy(col_hbm.at[wid], cidx_v)
    pltpu.sync_copy(ones_hbm, ones_v)
    pltpu.sync_copy(
        z16_hbm.at[pl.ds(sid * RPT, RPT)], deg_sh.at[pl.ds(sid * RPT, RPT)]
    )
    plsc.subcore_barrier()

    @pl.loop(0, DNCH)
    def _(j):
        pltpu.sync_copy(ones_v, deg_sh.at[cidx_v.at[j]], add=True)

    plsc.subcore_barrier()
    pltpu.sync_copy(
        deg_sh.at[pl.ds(sid * RPT, RPT)],
        out_hbm.at[core, pl.ds(sid * RPT, RPT)],
    )


_deg_kernel = pl.kernel(
    _deg_body,
    out_type=jax.ShapeDtypeStruct((NC, NPAD, LANES), jnp.float32),
    mesh=_mesh,
    compiler_params=pltpu.CompilerParams(use_tc_tiling_on_sc=False),
    scratch_types=[
        pltpu.VMEM((DNCH, DK), jnp.int32),
        pltpu.VMEM((DK, LANES), jnp.float32),
        pltpu.VMEM_SHARED((NPAD, LANES), jnp.float32),
    ],
)


def _scatter_body(g_hbm, row_hbm, col_hbm, z_hbm, out_hbm,
                  ridx_v, cidx_v, rows_v, agg_sh, *sems):
    gsems, ssems = sems[:R], sems[R:]
    core = lax.axis_index("c")
    sid = lax.axis_index("s")
    wid = sid * NC + core

    pltpu.sync_copy(
        z_hbm.at[pl.ds(sid * RPT, RPT)], agg_sh.at[pl.ds(sid * RPT, RPT)]
    )
    plsc.subcore_barrier()

    for p in range(PH):
        pltpu.sync_copy(row_hbm.at[wid, pl.ds(p * CHP, CHP)], ridx_v)
        pltpu.sync_copy(col_hbm.at[wid, pl.ds(p * CHP, CHP)], cidx_v)

        @pl.loop(0, NBR)
        def _(t):
            base = t * R
            for r in range(R):
                @pl.when(t > 0)
                def _():
                    pltpu.make_async_copy(
                        rows_v.at[r], agg_sh.at[cidx_v.at[base - R + r]],
                        ssems[r],
                    ).wait()

                pltpu.async_copy(
                    g_hbm.at[ridx_v.at[base + r]], rows_v.at[r], gsems[r]
                )
            for r in range(R):
                pltpu.make_async_copy(
                    g_hbm.at[ridx_v.at[base + r]], rows_v.at[r], gsems[r]
                ).wait()
                pltpu.async_copy(
                    rows_v.at[r], agg_sh.at[cidx_v.at[base + r]], ssems[r],
                    add=True,
                )

        for r in range(R):
            pltpu.make_async_copy(
                rows_v.at[r], agg_sh.at[cidx_v.at[CHP - R + r]], ssems[r]
            ).wait()

    plsc.subcore_barrier()
    pltpu.sync_copy(
        agg_sh.at[pl.ds(sid * RPT, RPT)],
        out_hbm.at[core, pl.ds(sid * RPT, RPT)],
    )


_scatter_kernel = pl.kernel(
    _scatter_body,
    out_type=jax.ShapeDtypeStruct((NC, NPAD, D), jnp.float32),
    mesh=_mesh,
    scratch_types=[
        pltpu.VMEM((CHP, K), jnp.int32),
        pltpu.VMEM((CHP, K), jnp.int32),
        pltpu.VMEM((R, K, D), jnp.float32),
        pltpu.VMEM_SHARED((NPAD, D), jnp.float32),
    ] + [pltpu.SemaphoreType.DMA] * (2 * R),
)


BN = 2000
_BETAS = tuple(float(math.log(THETA / (i + 1) + 1.0)) for i in range(L))


def _pre_body(x_ref, w2_ref, *p_refs):
    x0 = ALPHA * jnp.maximum(x_ref[...], 0.0)
    for i in range(L):
        b = _BETAS[i]
        p_refs[i][...] = (1.0 - b) * x0 + b * jnp.dot(
            x0, w2_ref[i], preferred_element_type=jnp.float32
        )


def _pre_call(x, W2):
    return pl.pallas_call(
        _pre_body,
        grid=(N // BN,),
        in_specs=[
            pl.BlockSpec((BN, D), lambda i: (i, 0)),
            pl.BlockSpec((L, D, D), lambda i: (0, 0, 0)),
        ],
        out_specs=[pl.BlockSpec((BN, D), lambda i: (i, 0))] * L,
        out_shape=[jax.ShapeDtypeStruct((N, D), jnp.float32)] * L,
    )(x, W2)


def _init_body(x_ref, degp_ref, f0_ref, g0_ref, dinvb_ref):
    degp = degp_ref[...]
    deg = degp[0, :, 0:1] + degp[1, :, 0:1] + 1.0
    dinvb = jnp.broadcast_to(lax.rsqrt(deg), (BN, D))
    f0 = jnp.maximum(x_ref[...], 0.0)
    f0_ref[...] = f0
    g0_ref[...] = f0 * dinvb
    dinvb_ref[...] = dinvb


def _init_call(x, degp):
    return pl.pallas_call(
        _init_body,
        grid=(N // BN,),
        in_specs=[
            pl.BlockSpec((BN, D), lambda i: (i, 0)),
            pl.BlockSpec((NC, BN, LANES), lambda i: (0, i, 0)),
        ],
        out_specs=[
            pl.BlockSpec((BN, D), lambda i: (i, 0)),
            pl.BlockSpec((BN, D), lambda i: (i, 0)),
            pl.BlockSpec((BN, D), lambda i: (i, 0)),
        ],
        out_shape=[
            jax.ShapeDtypeStruct((N, D), jnp.float32),
            jax.ShapeDtypeStruct((N, D), jnp.float32),
            jax.ShapeDtypeStruct((N, D), jnp.float32),
        ],
    )(x, degp)


def _dense_body(beta, s_ref, g_ref, p_ref, dinvb_ref, w1_ref, f_ref, gn_ref):
    dinvb = dinvb_ref[...]
    agg = dinvb * (s_ref[0] + s_ref[1] + g_ref[...])
    h = (1.0 - ALPHA) * agg
    out = (1.0 - beta) * h + beta * jnp.dot(
        h, w1_ref[...], preferred_element_type=jnp.float32
    ) + p_ref[...]
    f = jnp.maximum(out, 0.0)
    f_ref[...] = f
    gn_ref[...] = f * dinvb


def _dense_call(beta, S, g, p, dinvb, w1):
    return pl.pallas_call(
        functools.partial(_dense_body, beta),
        grid=(N // BN,),
        in_specs=[
            pl.BlockSpec((NC, BN, D), lambda i: (0, i, 0)),
            pl.BlockSpec((BN, D), lambda i: (i, 0)),
            pl.BlockSpec((BN, D), lambda i: (i, 0)),
            pl.BlockSpec((BN, D), lambda i: (i, 0)),
            pl.BlockSpec((D, D), lambda i: (0, 0)),
        ],
        out_specs=[
            pl.BlockSpec((BN, D), lambda i: (i, 0)),
            pl.BlockSpec((BN, D), lambda i: (i, 0)),
        ],
        out_shape=[
            jax.ShapeDtypeStruct((N, D), jnp.float32),
            jax.ShapeDtypeStruct((N, D), jnp.float32),
        ],
    )(S, g, p, dinvb, w1)


def kernel(x, edge_index, W1, W2):
    row_r = edge_index[0].reshape(NW, EPW)
    col_r = edge_index[1].reshape(NW, EPW)
    fi = lax.broadcasted_iota(jnp.int32, (NW, EPT - EPW), 1)
    fake_row = (fi * 41) % N
    fake_col = N + fi % (NPAD - N)
    row3 = jnp.concatenate([row_r, fake_row], axis=1).reshape(NW, CH, K)
    col3 = jnp.concatenate([col_r, fake_col], axis=1).reshape(NW, CH, K)
    z16 = jnp.zeros((NPAD, LANES), jnp.float32)
    z128 = jnp.zeros((NPAD, D), jnp.float32)

    col2 = edge_index[1].reshape(NW, DNCH, DK)
    ones16 = jnp.ones((DK, LANES), jnp.float32)
    degp = _deg_kernel(col2, ones16, z16)
    P = _pre_call(x, W2)
    f0, g, dinvb = _init_call(x, degp)
    f = f0
    for i in range(L):
        S = _scatter_kernel(g, row3, col3, z128)
        f, g = _dense_call(_BETAS[i], S, g, P[i], dinvb, W1[i])
    return f


# speedup vs baseline: 22.0171x; 1.1688x over previous
import functools
import math

import jax
import jax.numpy as jnp
from jax import lax
from jax.experimental import pallas as pl
from jax.experimental.pallas import tpu as pltpu
from jax.experimental.pallas import tpu_sc as plsc

N = 10000
D = 128
E = 320000
L = 4
ALPHA = 0.5
THETA = 1.0

NC = 2
NS = 16
LANES = 16
NW = NC * NS
EPW = E // NW
K = 80
CH = 128
EPT = CH * K
PH = 4
CHP = CH // PH
R = 4
NBR = CHP // R
NPAD = 10112
RPT = NPAD // NS

_mesh = plsc.VectorSubcoreMesh(
    core_axis_name="c", subcore_axis_name="s", num_cores=NC, num_subcores=NS
)


DK = 80
DNCH = EPW // DK


def _deg_body(col_hbm, ones_hbm, z16_hbm, out_hbm, cidx_v, ones_v, deg_sh):
    core = lax.axis_index("c")
    sid = lax.axis_index("s")
    wid = sid * NC + core

    pltpu.sync_copy(col_hbm.at[wid], cidx_v)
    pltpu.sync_copy(ones_hbm, ones_v)
    pltpu.sync_copy(
        z16_hbm.at[pl.ds(sid * RPT, RPT)], deg_sh.at[pl.ds(sid * RPT, RPT)]
    )
    plsc.subcore_barrier()

    @pl.loop(0, DNCH)
    def _(j):
        pltpu.sync_copy(ones_v, deg_sh.at[cidx_v.at[j]], add=True)

    plsc.subcore_barrier()
    pltpu.sync_copy(
        deg_sh.at[pl.ds(sid * RPT, RPT)],
        out_hbm.at[core, pl.ds(sid * RPT, RPT)],
    )


_deg_kernel = pl.kernel(
    _deg_body,
    out_type=jax.ShapeDtypeStruct((NC, NPAD, LANES), jnp.float32),
    mesh=_mesh,
    compiler_params=pltpu.CompilerParams(use_tc_tiling_on_sc=False),
    scratch_types=[
        pltpu.VMEM((DNCH, DK), jnp.int32),
        pltpu.VMEM((DK, LANES), jnp.float32),
        pltpu.VMEM_SHARED((NPAD, LANES), jnp.float32),
    ],
)


def _scatter_body(g_hbm, row_hbm, col_hbm, z_hbm, out_hbm,
                  ridx_v, cidx_v, rows_v, agg_sh, *sems):
    gsems, ssems = sems[:R], sems[R:]
    core = lax.axis_index("c")
    sid = lax.axis_index("s")
    wid = sid * NC + core

    pltpu.sync_copy(
        z_hbm.at[pl.ds(sid * RPT, RPT)], agg_sh.at[pl.ds(sid * RPT, RPT)]
    )
    plsc.subcore_barrier()

    for p in range(PH):
        pltpu.sync_copy(row_hbm.at[wid, pl.ds(p * CHP, CHP)], ridx_v)
        pltpu.sync_copy(col_hbm.at[wid, pl.ds(p * CHP, CHP)], cidx_v)

        @pl.loop(0, NBR)
        def _(t):
            base = t * R
            for r in range(R):
                @pl.when(t > 0)
                def _():
                    pltpu.make_async_copy(
                        rows_v.at[r], agg_sh.at[cidx_v.at[base - R + r]],
                        ssems[r],
                    ).wait()

                pltpu.async_copy(
                    g_hbm.at[ridx_v.at[base + r]], rows_v.at[r], gsems[r]
                )
            for r in range(R):
                pltpu.make_async_copy(
                    g_hbm.at[ridx_v.at[base + r]], rows_v.at[r], gsems[r]
                ).wait()
                pltpu.async_copy(
                    rows_v.at[r], agg_sh.at[cidx_v.at[base + r]], ssems[r],
                    add=True,
                )

        for r in range(R):
            pltpu.make_async_copy(
                rows_v.at[r], agg_sh.at[cidx_v.at[CHP - R + r]], ssems[r]
            ).wait()

    plsc.subcore_barrier()
    pltpu.sync_copy(
        agg_sh.at[pl.ds(sid * RPT, RPT)],
        out_hbm.at[core, pl.ds(sid * RPT, RPT)],
    )


_scatter_kernel = pl.kernel(
    _scatter_body,
    out_type=jax.ShapeDtypeStruct((NC, NPAD, D), jnp.float32),
    mesh=_mesh,
    scratch_types=[
        pltpu.VMEM((CHP, K), jnp.int32),
        pltpu.VMEM((CHP, K), jnp.int32),
        pltpu.VMEM((R, K, D), jnp.float32),
        pltpu.VMEM_SHARED((NPAD, D), jnp.float32),
    ] + [pltpu.SemaphoreType.DMA] * (2 * R),
)


BN = 2000
_BETAS = tuple(float(math.log(THETA / (i + 1) + 1.0)) for i in range(L))


def _pre_body(x_ref, w2_ref, *p_refs):
    x0 = ALPHA * jnp.maximum(x_ref[...], 0.0)
    for i in range(L):
        b = _BETAS[i]
        p_refs[i][...] = (1.0 - b) * x0 + b * jnp.dot(
            x0, w2_ref[i], preferred_element_type=jnp.float32
        )


def _pre_call(x, W2):
    return pl.pallas_call(
        _pre_body,
        grid=(N // BN,),
        in_specs=[
            pl.BlockSpec((BN, D), lambda i: (i, 0)),
            pl.BlockSpec((L, D, D), lambda i: (0, 0, 0)),
        ],
        out_specs=[pl.BlockSpec((BN, D), lambda i: (i, 0))] * L,
        out_shape=[jax.ShapeDtypeStruct((N, D), jnp.float32)] * L,
    )(x, W2)


def _init_body(x_ref, degp_ref, f0_ref, g0_ref, dinvb_ref):
    degp = degp_ref[...]
    deg = degp[0, :, 0:1] + degp[1, :, 0:1] + 1.0
    dinvb = jnp.broadcast_to(lax.rsqrt(deg), (BN, D))
    f0 = jnp.maximum(x_ref[...], 0.0)
    f0_ref[...] = f0
    g0_ref[...] = f0 * dinvb
    dinvb_ref[...] = dinvb


def _init_call(x, degp):
    return pl.pallas_call(
        _init_body,
        grid=(N // BN,),
        in_specs=[
            pl.BlockSpec((BN, D), lambda i: (i, 0)),
            pl.BlockSpec((NC, BN, LANES), lambda i: (0, i, 0)),
        ],
        out_specs=[
            pl.BlockSpec((BN, D), lambda i: (i, 0)),
            pl.BlockSpec((BN, D), lambda i: (i, 0)),
            pl.BlockSpec((BN, D), lambda i: (i, 0)),
        ],
        out_shape=[
            jax.ShapeDtypeStruct((N, D), jnp.float32),
            jax.ShapeDtypeStruct((N, D), jnp.float32),
            jax.ShapeDtypeStruct((N, D), jnp.float32),
        ],
    )(x, degp)


def _dense_body(beta, s_ref, g_ref, p_ref, dinvb_ref, w1_ref, f_ref, gn_ref):
    dinvb = dinvb_ref[...]
    agg = dinvb * (s_ref[0] + s_ref[1] + g_ref[...])
    h = (1.0 - ALPHA) * agg
    out = (1.0 - beta) * h + beta * jnp.dot(
        h, w1_ref[...], preferred_element_type=jnp.float32
    ) + p_ref[...]
    f = jnp.maximum(out, 0.0)
    f_ref[...] = f
    gn_ref[...] = f * dinvb


def _dense_call(beta, S, g, p, dinvb, w1):
    return pl.pallas_call(
        functools.partial(_dense_body, beta),
        grid=(N // BN,),
        in_specs=[
            pl.BlockSpec((NC, BN, D), lambda i: (0, i, 0)),
            pl.BlockSpec((BN, D), lambda i: (i, 0)),
            pl.BlockSpec((BN, D), lambda i: (i, 0)),
            pl.BlockSpec((BN, D), lambda i: (i, 0)),
            pl.BlockSpec((D, D), lambda i: (0, 0)),
        ],
        out_specs=[
            pl.BlockSpec((BN, D), lambda i: (i, 0)),
            pl.BlockSpec((BN, D), lambda i: (i, 0)),
        ],
        out_shape=[
            jax.ShapeDtypeStruct((N, D), jnp.float32),
            jax.ShapeDtypeStruct((N, D), jnp.float32),
        ],
    )(S, g, p, dinvb, w1)


def kernel(x, edge_index, W1, W2):
    row_r = edge_index[0].reshape(NW, EPW)
    col_r = edge_index[1].reshape(NW, EPW)
    fi = lax.broadcasted_iota(jnp.int32, (NW, EPT - EPW), 1)
    fake_row = (fi * 41) % N
    fake_col = N + fi % (NPAD - N)
    row3 = jnp.concatenate([row_r, fake_row], axis=1).reshape(NW, CH, K)
    col3 = jnp.concatenate([col_r, fake_col], axis=1).reshape(NW, CH, K)
    z16 = jnp.zeros((NPAD, LANES), jnp.float32)
    z128 = jnp.zeros((NPAD, D), jnp.float32)

    col2 = edge_index[1].reshape(NW, DNCH, DK)
    ones16 = jnp.ones((DK, LANES), jnp.float32)
    degp = _deg_kernel(col2, ones16, z16)
    P = _pre_call(x, W2)
    f0, g, dinvb = _init_call(x, degp)
    f = f0
    for i in range(L):
        S = _scatter_kernel(g, row3, col3, z128)
        f, g = _dense_call(_BETAS[i], S, g, P[i], dinvb, W1[i])
    return f
